# Initial kernel scaffold; baseline (speedup 1.0000x reference)
import jax
import jax.numpy as jnp
from jax.experimental import pallas as pl


def kernel(x, edge_index, W1, b1, W2, b2):
    raise NotImplementedError("write your pallas kernel here")


# speedup vs baseline: 22.1466x; 22.1466x over previous
import functools

import jax
import jax.numpy as jnp
from jax import lax
from jax.experimental import pallas as pl
from jax.experimental.pallas import tpu as pltpu
from jax.experimental.pallas import tpu_sc as plsc

_N = 10000
_E = 320000
_D = 128
_DH = _D // 2
_NC = 2
_NS = 16
_NPAD = 10240
_RPT = _NPAD // _NS
_CH = 80
_NBUF = 2
_DEGW = 16
_ROWBLK = 1000

_EPT_DEG = _E // (_NC * _NS)
_NCHUNK_DEG = _EPT_DEG // _CH
_EPT_AGG = _E // _NS
_NCHUNK_AGG = _EPT_AGG // _CH
_NOUTER_AGG = _NCHUNK_AGG // _NBUF

_mesh = plsc.VectorSubcoreMesh(core_axis_name="c", subcore_axis_name="s")


@functools.partial(
    pl.kernel,
    out_type=jax.ShapeDtypeStruct((_NC, _NPAD, _DEGW), jnp.float32),
    mesh=_mesh,
    scratch_types=[
        pltpu.VMEM((_NCHUNK_DEG, _CH), jnp.int32),
        pltpu.VMEM((_CH, _DEGW), jnp.float32),
        pltpu.VMEM_SHARED((_NPAD, _DEGW), jnp.float32),
    ],
    compiler_params=pltpu.CompilerParams(use_tc_tiling_on_sc=False),
)
def _deg_sc(dst_hbm, ones_hbm, zeros_hbm, out_hbm, dst_v, ones_v, shared_deg):
    cid = lax.axis_index("c")
    sid = lax.axis_index("s")
    pltpu.sync_copy(zeros_hbm, shared_deg.at[pl.ds(sid * _RPT, _RPT)])
    pltpu.sync_copy(dst_hbm.at[cid, sid], dst_v)
    pltpu.sync_copy(ones_hbm, ones_v)
    plsc.subcore_barrier()

    def body(j, carry):
        pltpu.sync_copy(ones_v, shared_deg.at[dst_v.at[j]], add=True)
        return carry

    lax.fori_loop(0, _NCHUNK_DEG, body, 0)
    plsc.subcore_barrier()
    pltpu.sync_copy(shared_deg.at[pl.ds(sid * _RPT, _RPT)],
                    out_hbm.at[cid, pl.ds(sid * _RPT, _RPT)])


@functools.partial(
    pl.kernel,
    out_type=jax.ShapeDtypeStruct((_NC, _NPAD, _DH), jnp.float32),
    mesh=_mesh,
    scratch_types=[
        pltpu.VMEM((_NCHUNK_AGG, _CH), jnp.int32),
        pltpu.VMEM((_NCHUNK_AGG, _CH), jnp.int32),
        pltpu.VMEM((_NBUF, _CH, _DH), jnp.float32),
        pltpu.VMEM_SHARED((_NPAD, _DH), jnp.float32),
    ] + [pltpu.SemaphoreType.DMA] * _NBUF,
    compiler_params=pltpu.CompilerParams(use_tc_tiling_on_sc=False),
)
def _agg_sc(src_hbm, dst_hbm, y_hbm, zeros_hbm, out_hbm,
            src_v, dst_v, rows_v, shared_agg, *sems):
    cid = lax.axis_index("c")
    sid = lax.axis_index("s")
    pltpu.sync_copy(zeros_hbm, shared_agg.at[pl.ds(sid * _RPT, _RPT)])
    pltpu.sync_copy(src_hbm.at[cid, sid], src_v)
    pltpu.sync_copy(dst_hbm.at[sid], dst_v)
    plsc.subcore_barrier()

    for b in range(_NBUF):
        pltpu.async_copy(y_hbm.at[src_v.at[b]], rows_v.at[b], sems[b])

    def body(g, carry):
        for b in range(_NBUF):
            j = g * _NBUF + b
            pltpu.make_async_copy(y_hbm.at[src_v.at[j]], rows_v.at[b],
                                  sems[b]).wait()
            pltpu.sync_copy(rows_v.at[b], shared_agg.at[dst_v.at[j]], add=True)

            @pl.when(j + _NBUF < _NCHUNK_AGG)
            def _():
                pltpu.async_copy(y_hbm.at[src_v.at[j + _NBUF]], rows_v.at[b],
                                 sems[b])
        return carry

    lax.fori_loop(0, _NOUTER_AGG, body, 0)
    plsc.subcore_barrier()
    pltpu.sync_copy(shared_agg.at[pl.ds(sid * _RPT, _RPT)],
                    out_hbm.at[cid, pl.ds(sid * _RPT, _RPT)])


def _dinv_rows(d0_ref, d1_ref):
    deg = d0_ref[:, 0:1] + d1_ref[:, 0:1] + 1.0
    return lax.rsqrt(deg)


def _split_halves(full, o_ref):
    o_ref[0] = full[:, :_DH]
    o_ref[1] = full[:, _DH:]


def _gather_halves(a_ref, y_ref):
    return jnp.concatenate(
        [a_ref[0] + y_ref[0], a_ref[1] + y_ref[1]], axis=1)


def _y1_tc(x_ref, w_ref, d0_ref, d1_ref, o_ref):
    dinv = _dinv_rows(d0_ref, d1_ref)
    xw = jnp.dot(x_ref[...], w_ref[...], preferred_element_type=jnp.float32)
    _split_halves(xw * dinv, o_ref)


def _mid_tc(a_ref, y_ref, b_ref, w_ref, d0_ref, d1_ref, o_ref):
    dinv = _dinv_rows(d0_ref, d1_ref)
    h = _gather_halves(a_ref, y_ref) * dinv + b_ref[...]
    h = jnp.maximum(h, 0.0)
    hw = jnp.dot(h, w_ref[...], preferred_element_type=jnp.float32)
    _split_halves(hw * dinv, o_ref)


def _out_tc(a_ref, y_ref, b_ref, d0_ref, d1_ref, o_ref):
    dinv = _dinv_rows(d0_ref, d1_ref)
    h = _gather_halves(a_ref, y_ref) * dinv + b_ref[...]
    o_ref[...] = jnp.maximum(h, 0.0)


_row_spec = pl.BlockSpec((_ROWBLK, _D), lambda i: (i, 0))
_half_spec = pl.BlockSpec((2, _ROWBLK, _DH), lambda i: (0, i, 0))
_deg_spec = pl.BlockSpec((_ROWBLK, _DEGW), lambda i: (i, 0))
_w_spec = pl.BlockSpec((_D, _D), lambda i: (0, 0))
_b_spec = pl.BlockSpec((1, _D), lambda i: (0, 0))
_grid = (_N // _ROWBLK,)
_out_full = jax.ShapeDtypeStruct((_N, _D), jnp.float32)
_out_half = jax.ShapeDtypeStruct((2, _N, _DH), jnp.float32)


def kernel(x, edge_index, W1, b1, W2, b2):
    src = edge_index[0].reshape(_NS, _NCHUNK_AGG, _CH)
    src2 = jnp.stack([src, src + _N])
    dst_deg = edge_index[1].reshape(_NC, _NS, _NCHUNK_DEG, _CH)
    dst_agg = edge_index[1].reshape(_NS, _NCHUNK_AGG, _CH)
    ones_deg = jnp.ones((_CH, _DEGW), jnp.float32)
    zeros_deg = jnp.zeros((_RPT, _DEGW), jnp.float32)
    zeros_row = jnp.zeros((_RPT, _DH), jnp.float32)
    b1r = b1.reshape(1, _D)
    b2r = b2.reshape(1, _D)

    deg = _deg_sc(dst_deg, ones_deg, zeros_deg)
    d0, d1 = deg[0], deg[1]

    y1 = pl.pallas_call(
        _y1_tc,
        grid=_grid,
        in_specs=[_row_spec, _w_spec, _deg_spec, _deg_spec],
        out_specs=_half_spec,
        out_shape=_out_half,
    )(x, W1, d0, d1)

    agg1 = _agg_sc(src2, dst_agg, y1.reshape(2 * _N, _DH), zeros_row)

    y2 = pl.pallas_call(
        _mid_tc,
        grid=_grid,
        in_specs=[_half_spec, _half_spec, _b_spec, _w_spec,
                  _deg_spec, _deg_spec],
        out_specs=_half_spec,
        out_shape=_out_half,
    )(agg1, y1, b1r, W2, d0, d1)

    agg2 = _agg_sc(src2, dst_agg, y2.reshape(2 * _N, _DH), zeros_row)

    out = pl.pallas_call(
        _out_tc,
        grid=_grid,
        in_specs=[_half_spec, _half_spec, _b_spec, _deg_spec, _deg_spec],
        out_specs=_row_spec,
        out_shape=_out_full,
    )(agg2, y2, b2r, d0, d1)

    return out


# speedup vs baseline: 29.7424x; 1.3430x over previous
import functools

import jax
import jax.numpy as jnp
from jax import lax
from jax.experimental import pallas as pl
from jax.experimental.pallas import tpu as pltpu
from jax.experimental.pallas import tpu_sc as plsc

_N = 10000
_E = 320000
_D = 128
_DH = _D // 2
_NC = 2
_NS = 16
_NPAD = 10240
_RPT = _NPAD // _NS
_CH = 80
_NBUF = 4
_DEGW = 16
_ROWBLK = 1000

_EPT_DEG = _E // (_NC * _NS)
_NCHUNK_DEG = _EPT_DEG // _CH
_EPT_AGG = _E // _NS
_NCHUNK_AGG = _EPT_AGG // _CH

_mesh = plsc.VectorSubcoreMesh(core_axis_name="c", subcore_axis_name="s")


@functools.partial(
    pl.kernel,
    out_type=jax.ShapeDtypeStruct((_NC, _NPAD, _DEGW), jnp.float32),
    mesh=_mesh,
    scratch_types=[
        pltpu.VMEM((_NCHUNK_DEG, _CH), jnp.int32),
        pltpu.VMEM((_CH, _DEGW), jnp.float32),
        pltpu.VMEM_SHARED((_NPAD, _DEGW), jnp.float32),
    ],
    compiler_params=pltpu.CompilerParams(use_tc_tiling_on_sc=False),
)
def _deg_sc(dst_hbm, ones_hbm, zeros_hbm, out_hbm, dst_v, ones_v, shared_deg):
    cid = lax.axis_index("c")
    sid = lax.axis_index("s")
    pltpu.sync_copy(zeros_hbm, shared_deg.at[pl.ds(sid * _RPT, _RPT)])
    pltpu.sync_copy(dst_hbm.at[cid, sid], dst_v)
    pltpu.sync_copy(ones_hbm, ones_v)
    plsc.subcore_barrier()

    def body(j, carry):
        pltpu.sync_copy(ones_v, shared_deg.at[dst_v.at[j]], add=True)
        return carry

    lax.fori_loop(0, _NCHUNK_DEG, body, 0)
    plsc.subcore_barrier()
    pltpu.sync_copy(shared_deg.at[pl.ds(sid * _RPT, _RPT)],
                    out_hbm.at[cid, pl.ds(sid * _RPT, _RPT)])


@functools.partial(
    pl.kernel,
    out_type=jax.ShapeDtypeStruct((_NC, _NPAD, _DH), jnp.float32),
    mesh=_mesh,
    scratch_types=[
        pltpu.VMEM((_NCHUNK_AGG, _CH), jnp.int32),
        pltpu.VMEM((_NCHUNK_AGG, _CH), jnp.int32),
        pltpu.VMEM((_NBUF, _CH, _DH), jnp.float32),
        pltpu.VMEM_SHARED((_NPAD, _DH), jnp.float32),
    ] + [pltpu.SemaphoreType.DMA] * (2 * _NBUF),
    compiler_params=pltpu.CompilerParams(use_tc_tiling_on_sc=False),
)
def _agg_sc(src_hbm, dst_hbm, y_hbm, zeros_hbm, out_hbm,
            src_v, dst_v, rows_v, shared_agg, *sems):
    cid = lax.axis_index("c")
    sid = lax.axis_index("s")
    gsem = sems[:_NBUF]
    ssem = sems[_NBUF:]
    pltpu.sync_copy(zeros_hbm, shared_agg.at[pl.ds(sid * _RPT, _RPT)])
    pltpu.sync_copy(src_hbm.at[cid, sid], src_v)
    pltpu.sync_copy(dst_hbm.at[sid], dst_v)
    plsc.subcore_barrier()

    def fire_g(j, b):
        pltpu.async_copy(y_hbm.at[src_v.at[j]], rows_v.at[b], gsem[b])

    def wait_g(j, b):
        pltpu.make_async_copy(y_hbm.at[src_v.at[j]], rows_v.at[b],
                              gsem[b]).wait()

    def fire_s(j, b):
        pltpu.async_copy(rows_v.at[b], shared_agg.at[dst_v.at[j]], ssem[b],
                         add=True)

    def wait_s(j, b):
        pltpu.make_async_copy(rows_v.at[b], shared_agg.at[dst_v.at[j]],
                              ssem[b]).wait()

    for b in range(_NBUF - 1):
        fire_g(b, b)

    def body(g, carry):
        for b in range(_NBUF):
            j = g * _NBUF + b

            @pl.when(j >= 1)
            def _():
                wait_s(j - 1, (b - 1) % _NBUF)

            @pl.when(j + _NBUF - 1 < _NCHUNK_AGG)
            def _():
                fire_g(j + _NBUF - 1, (b - 1) % _NBUF)

            wait_g(j, b)
            fire_s(j, b)
        return carry

    lax.fori_loop(0, _NCHUNK_AGG // _NBUF, body, 0)

    rem_start = (_NCHUNK_AGG // _NBUF) * _NBUF
    for j in range(rem_start, _NCHUNK_AGG):
        b = j % _NBUF
        wait_s(j - 1, (b - 1) % _NBUF)
        wait_g(j, b)
        fire_s(j, b)
    wait_s(_NCHUNK_AGG - 1, (_NCHUNK_AGG - 1) % _NBUF)
    plsc.subcore_barrier()
    pltpu.sync_copy(shared_agg.at[pl.ds(sid * _RPT, _RPT)],
                    out_hbm.at[cid, pl.ds(sid * _RPT, _RPT)])


def _dinv_rows(d0_ref, d1_ref):
    deg = d0_ref[:, 0:1] + d1_ref[:, 0:1] + 1.0
    return lax.rsqrt(deg)


def _split_halves(full, o_ref):
    o_ref[0] = full[:, :_DH]
    o_ref[1] = full[:, _DH:]


def _gather_halves(a_ref, y_ref):
    return jnp.concatenate(
        [a_ref[0] + y_ref[0], a_ref[1] + y_ref[1]], axis=1)


def _y1_tc(x_ref, w_ref, d0_ref, d1_ref, o_ref):
    dinv = _dinv_rows(d0_ref, d1_ref)
    xw = jnp.dot(x_ref[...], w_ref[...], preferred_element_type=jnp.float32)
    _split_halves(xw * dinv, o_ref)


def _mid_tc(a_ref, y_ref, b_ref, w_ref, d0_ref, d1_ref, o_ref):
    dinv = _dinv_rows(d0_ref, d1_ref)
    h = _gather_halves(a_ref, y_ref) * dinv + b_ref[...]
    h = jnp.maximum(h, 0.0)
    hw = jnp.dot(h, w_ref[...], preferred_element_type=jnp.float32)
    _split_halves(hw * dinv, o_ref)


def _out_tc(a_ref, y_ref, b_ref, d0_ref, d1_ref, o_ref):
    dinv = _dinv_rows(d0_ref, d1_ref)
    h = _gather_halves(a_ref, y_ref) * dinv + b_ref[...]
    o_ref[...] = jnp.maximum(h, 0.0)


_row_spec = pl.BlockSpec((_ROWBLK, _D), lambda i: (i, 0))
_half_spec = pl.BlockSpec((2, _ROWBLK, _DH), lambda i: (0, i, 0))
_deg_spec = pl.BlockSpec((_ROWBLK, _DEGW), lambda i: (i, 0))
_w_spec = pl.BlockSpec((_D, _D), lambda i: (0, 0))
_b_spec = pl.BlockSpec((1, _D), lambda i: (0, 0))
_grid = (_N // _ROWBLK,)
_out_full = jax.ShapeDtypeStruct((_N, _D), jnp.float32)
_out_half = jax.ShapeDtypeStruct((2, _N, _DH), jnp.float32)


def kernel(x, edge_index, W1, b1, W2, b2):
    src = edge_index[0].reshape(_NS, _NCHUNK_AGG, _CH)
    src2 = jnp.stack([src, src + _N])
    dst_deg = edge_index[1].reshape(_NC, _NS, _NCHUNK_DEG, _CH)
    dst_agg = edge_index[1].reshape(_NS, _NCHUNK_AGG, _CH)
    ones_deg = jnp.ones((_CH, _DEGW), jnp.float32)
    zeros_deg = jnp.zeros((_RPT, _DEGW), jnp.float32)
    zeros_row = jnp.zeros((_RPT, _DH), jnp.float32)
    b1r = b1.reshape(1, _D)
    b2r = b2.reshape(1, _D)

    deg = _deg_sc(dst_deg, ones_deg, zeros_deg)
    d0, d1 = deg[0], deg[1]

    y1 = pl.pallas_call(
        _y1_tc,
        grid=_grid,
        in_specs=[_row_spec, _w_spec, _deg_spec, _deg_spec],
        out_specs=_half_spec,
        out_shape=_out_half,
    )(x, W1, d0, d1)

    agg1 = _agg_sc(src2, dst_agg, y1.reshape(2 * _N, _DH), zeros_row)

    y2 = pl.pallas_call(
        _mid_tc,
        grid=_grid,
        in_specs=[_half_spec, _half_spec, _b_spec, _w_spec,
                  _deg_spec, _deg_spec],
        out_specs=_half_spec,
        out_shape=_out_half,
    )(agg1, y1, b1r, W2, d0, d1)

    agg2 = _agg_sc(src2, dst_agg, y2.reshape(2 * _N, _DH), zeros_row)

    out = pl.pallas_call(
        _out_tc,
        grid=_grid,
        in_specs=[_half_spec, _half_spec, _b_spec, _deg_spec, _deg_spec],
        out_specs=_row_spec,
        out_shape=_out_full,
    )(agg2, y2, b2r, d0, d1)

    return out
